# Initial kernel scaffold; baseline (speedup 1.0000x reference)
import jax
import jax.numpy as jnp
from jax.experimental import pallas as pl


def kernel(x, y):
    raise NotImplementedError("write your pallas kernel here")


# speedup vs baseline: 1.4862x; 1.4862x over previous
import functools

import jax
import jax.numpy as jnp
from jax import lax
from jax.experimental import pallas as pl
from jax.experimental.pallas import tpu as pltpu

_C = 21
_CP = 24
_SMOOTH = 1e-06
_R = 64
_W = 512


def _iou_body(x_ref, y_ref, o_ref, iacc, pacc, yacc, *, nb, b_total):
    b = pl.program_id(0)
    j = pl.program_id(1)

    @pl.when((b == 0) & (j == 0))
    def _init():
        iacc[...] = jnp.zeros_like(iacc)
        pacc[...] = jnp.zeros_like(pacc)
        yacc[...] = jnp.zeros_like(yacc)

    xb = x_ref[0]
    yb = y_ref[0]

    m = xb[0]
    pred = jnp.zeros((_R, _W), jnp.int32)
    for c in range(1, _C):
        xc = xb[c]
        gt = xc > m
        m = jnp.where(gt, xc, m)
        pred = jnp.where(gt, c, pred)

    match = pred == yb

    cls = lax.broadcasted_iota(jnp.int32, (_CP, _R, _W), 0)
    eqp = (pred[None, :, :] == cls)
    eqy = (yb[None, :, :] == cls)
    inter = eqy & match[None, :, :]

    def _count(e):
        s1 = jnp.sum(e.astype(jnp.float32), axis=2)
        return jnp.sum(s1, axis=1, keepdims=True)

    ic = _count(inter)
    pc = _count(eqp)
    yc = _count(eqy)

    iacc[b, :, 0:1] = iacc[b, :, 0:1] + ic
    pacc[b, :, 0:1] = pacc[b, :, 0:1] + pc
    yacc[b, :, 0:1] = yacc[b, :, 0:1] + yc

    @pl.when((b == b_total - 1) & (j == nb - 1))
    def _finalize():
        i = iacc[:, :, 0]
        p = pacc[:, :, 0]
        yv = yacc[:, :, 0]
        u = p + yv - i
        iou = (i + _SMOOTH) / (u + _SMOOTH)
        cmask = lax.broadcasted_iota(jnp.int32, (b_total, _CP), 1) < _C
        miou = jnp.sum(jnp.where(cmask, iou, 0.0), axis=1, keepdims=True) / _C
        loss = 1.0 - miou
        o_ref[...] = (jnp.sum(loss) / b_total).reshape(1, 1)


@jax.jit
def kernel(x, y):
    B, C, H, W = x.shape
    y = y.astype(jnp.int32)
    nb = H // _R
    out = pl.pallas_call(
        functools.partial(_iou_body, nb=nb, b_total=B),
        grid=(B, nb),
        in_specs=[
            pl.BlockSpec((1, C, _R, W), lambda b, j: (b, 0, j, 0)),
            pl.BlockSpec((1, _R, W), lambda b, j: (b, j, 0)),
        ],
        out_specs=pl.BlockSpec((1, 1), lambda b, j: (0, 0)),
        out_shape=jax.ShapeDtypeStruct((1, 1), jnp.float32),
        scratch_shapes=[
            pltpu.VMEM((B, _CP, 128), jnp.float32),
            pltpu.VMEM((B, _CP, 128), jnp.float32),
            pltpu.VMEM((B, _CP, 128), jnp.float32),
        ],
        compiler_params=pltpu.CompilerParams(
            dimension_semantics=("arbitrary", "arbitrary"),
        ),
    )(x, y)
    return out[0, 0]


# speedup vs baseline: 1.6059x; 1.0805x over previous
import functools

import jax
import jax.numpy as jnp
from jax import lax
from jax.experimental import pallas as pl
from jax.experimental.pallas import tpu as pltpu

_C = 21
_CP = 24
_SMOOTH = 1e-06
_R = 64
_W = 512


def _treesum(e):
    t = e[:, 0:32] + e[:, 32:64]
    t = t[:, 0:16] + t[:, 16:32]
    return t[:, 0:8] + t[:, 8:16]


def _iou_body(x_ref, y_ref, o_ref, iacc, pacc, yacc, *, nb, b_total):
    b = pl.program_id(0)
    j = pl.program_id(1)

    @pl.when((b == 0) & (j == 0))
    def _init():
        iacc[...] = jnp.zeros_like(iacc)
        pacc[...] = jnp.zeros_like(pacc)
        yacc[...] = jnp.zeros_like(yacc)

    xb = x_ref[0]
    yb = y_ref[0]

    m = xb[0]
    pred = jnp.zeros((_R, _W), jnp.int32)
    for c in range(1, _C):
        xc = xb[c]
        gt = xc > m
        m = jnp.where(gt, xc, m)
        pred = jnp.where(gt, c, pred)

    cls = lax.broadcasted_iota(jnp.int32, (_CP, _R, _W), 0)
    pf = jnp.where(pred[None, :, :] == cls, 1.0, 0.0)
    yf = jnp.where(yb[None, :, :] == cls, 1.0, 0.0)

    iacc[b] = iacc[b] + _treesum(pf * yf)
    pacc[b] = pacc[b] + _treesum(pf)
    yacc[b] = yacc[b] + _treesum(yf)

    @pl.when((b == b_total - 1) & (j == nb - 1))
    def _finalize():
        i = jnp.sum(iacc[...], axis=(2, 3))
        p = jnp.sum(pacc[...], axis=(2, 3))
        yv = jnp.sum(yacc[...], axis=(2, 3))
        u = p + yv - i
        iou = (i + _SMOOTH) / (u + _SMOOTH)
        cmask = lax.broadcasted_iota(jnp.int32, (b_total, _CP), 1) < _C
        miou = jnp.sum(jnp.where(cmask, iou, 0.0), axis=1, keepdims=True) / _C
        loss = 1.0 - miou
        o_ref[...] = (jnp.sum(loss) / b_total).reshape(1, 1)


@jax.jit
def kernel(x, y):
    B, C, H, W = x.shape
    y = y.astype(jnp.int32)
    nb = H // _R
    out = pl.pallas_call(
        functools.partial(_iou_body, nb=nb, b_total=B),
        grid=(B, nb),
        in_specs=[
            pl.BlockSpec((1, C, _R, W), lambda b, j: (b, 0, j, 0)),
            pl.BlockSpec((1, _R, W), lambda b, j: (b, j, 0)),
        ],
        out_specs=pl.BlockSpec((1, 1), lambda b, j: (0, 0)),
        out_shape=jax.ShapeDtypeStruct((1, 1), jnp.float32),
        scratch_shapes=[
            pltpu.VMEM((B, _CP, 8, W), jnp.float32),
            pltpu.VMEM((B, _CP, 8, W), jnp.float32),
            pltpu.VMEM((B, _CP, 8, W), jnp.float32),
        ],
        compiler_params=pltpu.CompilerParams(
            dimension_semantics=("arbitrary", "arbitrary"),
        ),
    )(x, y)
    return out[0, 0]


# speedup vs baseline: 1.6927x; 1.0540x over previous
import functools

import jax
import jax.numpy as jnp
from jax import lax
from jax.experimental import pallas as pl
from jax.experimental.pallas import tpu as pltpu

_C = 21
_CP = 24
_SMOOTH = 1e-06
_R = 64
_W = 512


def _treesum(e):
    t = e[0:32] + e[32:64]
    t = t[0:16] + t[16:32]
    return t[0:8] + t[8:16]


def _iou_body(x_ref, y_ref, o_ref, iacc, pacc, yacc, *, nb, b_total):
    b = pl.program_id(0)
    j = pl.program_id(1)

    @pl.when((b == 0) & (j == 0))
    def _init():
        iacc[...] = jnp.zeros_like(iacc)
        pacc[...] = jnp.zeros_like(pacc)
        yacc[...] = jnp.zeros_like(yacc)

    xb = x_ref[0]
    yb = y_ref[0]

    m = xb[0]
    pred = jnp.zeros((_R, _W), jnp.int32)
    for c in range(1, _C):
        xc = xb[c]
        gt = xc > m
        m = jnp.where(gt, xc, m)
        pred = jnp.where(gt, c, pred)

    for c in range(_C):
        pf = jnp.where(pred == c, 1.0, 0.0)
        yf = jnp.where(yb == c, 1.0, 0.0)
        iacc[b, c] = iacc[b, c] + _treesum(pf * yf)
        pacc[b, c] = pacc[b, c] + _treesum(pf)
        yacc[b, c] = yacc[b, c] + _treesum(yf)

    @pl.when((b == b_total - 1) & (j == nb - 1))
    def _finalize():
        i = jnp.sum(iacc[...], axis=(2, 3))
        p = jnp.sum(pacc[...], axis=(2, 3))
        yv = jnp.sum(yacc[...], axis=(2, 3))
        u = p + yv - i
        iou = (i + _SMOOTH) / (u + _SMOOTH)
        miou = jnp.sum(iou, axis=1, keepdims=True) / _C
        loss = 1.0 - miou
        o_ref[...] = (jnp.sum(loss) / b_total).reshape(1, 1)


@jax.jit
def kernel(x, y):
    B, C, H, W = x.shape
    y = y.astype(jnp.int32)
    nb = H // _R
    out = pl.pallas_call(
        functools.partial(_iou_body, nb=nb, b_total=B),
        grid=(B, nb),
        in_specs=[
            pl.BlockSpec((1, C, _R, W), lambda b, j: (b, 0, j, 0)),
            pl.BlockSpec((1, _R, W), lambda b, j: (b, j, 0)),
        ],
        out_specs=pl.BlockSpec((1, 1), lambda b, j: (0, 0)),
        out_shape=jax.ShapeDtypeStruct((1, 1), jnp.float32),
        scratch_shapes=[
            pltpu.VMEM((B, _C, 8, W), jnp.float32),
            pltpu.VMEM((B, _C, 8, W), jnp.float32),
            pltpu.VMEM((B, _C, 8, W), jnp.float32),
        ],
        compiler_params=pltpu.CompilerParams(
            dimension_semantics=("arbitrary", "arbitrary"),
        ),
    )(x, y)
    return out[0, 0]


# speedup vs baseline: 1.7791x; 1.0511x over previous
import functools

import jax
import jax.numpy as jnp
from jax import lax
from jax.experimental import pallas as pl
from jax.experimental.pallas import tpu as pltpu

_C = 21
_CP = 24
_SMOOTH = 1e-06
_R = 64
_W = 512


def _treesum(e):
    t = e[0:32] + e[32:64]
    t = t[0:16] + t[16:32]
    return t[0:8] + t[8:16]


def _iou_body(x_ref, y_ref, o_ref, acc, *, nb, b_total):
    b = pl.program_id(0)
    j = pl.program_id(1)

    @pl.when((b == 0) & (j == 0))
    def _init():
        acc[...] = jnp.zeros_like(acc)

    xb = x_ref[0]
    yb = y_ref[0]

    m = xb[0]
    pred = jnp.zeros((_R, _W), jnp.int32)
    for c in range(1, _C):
        xc = xb[c]
        gt = xc > m
        m = jnp.where(gt, xc, m)
        pred = jnp.where(gt, c, pred)

    for c in range(_C):
        pm = pred == c
        ym = yb == c
        s1 = jnp.where(ym, 16513.0, 128.0)
        s2 = jnp.where(ym, 16384.0, 0.0)
        acc[b, c] = acc[b, c] + _treesum(jnp.where(pm, s1, s2))

    @pl.when((b == b_total - 1) & (j == nb - 1))
    def _finalize():
        av = acc[...]
        yv_e = jnp.floor(av / 16384.0)
        rem = av - yv_e * 16384.0
        p_e = jnp.floor(rem / 128.0)
        i_e = rem - p_e * 128.0
        i = jnp.sum(i_e, axis=(2, 3))
        p = jnp.sum(p_e, axis=(2, 3))
        yv = jnp.sum(yv_e, axis=(2, 3))
        u = p + yv - i
        iou = (i + _SMOOTH) / (u + _SMOOTH)
        miou = jnp.sum(iou, axis=1, keepdims=True) / _C
        loss = 1.0 - miou
        o_ref[...] = (jnp.sum(loss) / b_total).reshape(1, 1)


@jax.jit
def kernel(x, y):
    B, C, H, W = x.shape
    y = y.astype(jnp.int32)
    nb = H // _R
    out = pl.pallas_call(
        functools.partial(_iou_body, nb=nb, b_total=B),
        grid=(B, nb),
        in_specs=[
            pl.BlockSpec((1, C, _R, W), lambda b, j: (b, 0, j, 0)),
            pl.BlockSpec((1, _R, W), lambda b, j: (b, j, 0)),
        ],
        out_specs=pl.BlockSpec((1, 1), lambda b, j: (0, 0)),
        out_shape=jax.ShapeDtypeStruct((1, 1), jnp.float32),
        scratch_shapes=[
            pltpu.VMEM((B, _C, 8, W), jnp.float32),
        ],
        compiler_params=pltpu.CompilerParams(
            dimension_semantics=("arbitrary", "arbitrary"),
        ),
    )(x, y)
    return out[0, 0]
